# baseline (speedup 1.0000x reference)
import jax
import jax.numpy as jnp
from jax import lax
from jax.experimental import pallas as pl
from jax.experimental.pallas import tpu as pltpu
from jax.experimental.pallas import tpu_sc as plsc

_B = 64
_T = 1024
_N = _B * _T
_NC = 2
_NS = 16
_NW = _NC * _NS
_CHUNK = _N // _NW
_BPW = _CHUNK // _T
_GB = 128
_NG = _CHUNK // _GB
_GPB = _T // _GB
_D = 64
_NB = 2
_NGRP = _NG // _NB


def _sc_body(h_hbm, w_hbm, tab_hbm, sin_out_hbm, cos_out_hbm,
             h_v, w_v, idx_v, rows, trs, trc, *sems):
    gsem = sems[0:_NB]
    wsem_s = sems[_NB:2 * _NB]
    wsem_c = sems[2 * _NB:3 * _NB]
    wid = lax.axis_index("s") * _NC + lax.axis_index("c")
    base = wid * _CHUNK
    b0 = wid * _BPW
    pltpu.sync_copy(h_hbm.at[pl.ds(base, _CHUNK)], h_v)
    pltpu.sync_copy(w_hbm.at[pl.ds(base, _CHUNK)], w_v)

    def idx_body(j, carry):
        for k in range(_GB // 16):
            t0 = j * _GB + k * 16
            idx_v[j, pl.ds(k * 16, 16)] = h_v[pl.ds(t0, 16)] * 32 + w_v[pl.ds(t0, 16)]
        return carry

    lax.fori_loop(0, _NG, idx_body, 0)

    def dsts(j):
        be = b0 + (j // _GPB)
        t0 = (j % _GPB) * _GB
        return (sin_out_hbm.at[be, :, pl.ds(t0, _GB)],
                cos_out_hbm.at[be, :, pl.ds(t0, _GB)])

    def fire_gather(j, b):
        pltpu.async_copy(tab_hbm.at[idx_v.at[j]], rows.at[b], gsem[b])

    for b in range(_NB):
        fire_gather(b, b)

    lanes = lax.iota(jnp.int32, 16)
    tvecs = [k * 16 + lanes for k in range(_GB // 16)]

    def transpose_batch(b):
        def d_body(d, carry):
            dv_s = jnp.broadcast_to(d, (16,))
            dv_c = dv_s + _D
            for k in range(_GB // 16):
                trs[b, d, pl.ds(k * 16, 16)] = plsc.load_gather(
                    rows.at[b], [tvecs[k], dv_s])
                trc[b, d, pl.ds(k * 16, 16)] = plsc.load_gather(
                    rows.at[b], [tvecs[k], dv_c])
            return carry

        lax.fori_loop(0, _D, d_body, 0)

    def group_body(g, carry):
        for b in range(_NB):
            j = g * _NB + b
            dst_s, dst_c = dsts(j)
            pltpu.make_async_copy(tab_hbm.at[idx_v.at[j]], rows.at[b],
                                  gsem[b]).wait()
            @pl.when(g > 0)
            def _():
                prev_s, prev_c = dsts(j - _NB)
                pltpu.make_async_copy(trs.at[b], prev_s, wsem_s[b]).wait()
                pltpu.make_async_copy(trc.at[b], prev_c, wsem_c[b]).wait()

            transpose_batch(b)
            @pl.when(g < _NGRP - 1)
            def _():
                fire_gather(j + _NB, b)

            pltpu.async_copy(trs.at[b], dst_s, wsem_s[b])
            pltpu.async_copy(trc.at[b], dst_c, wsem_c[b])

        return carry

    lax.fori_loop(0, _NGRP, group_body, 0)

    for b in range(_NB):
        j = (_NGRP - 1) * _NB + b
        dst_s, dst_c = dsts(j)
        pltpu.make_async_copy(trs.at[b], dst_s, wsem_s[b]).wait()
        pltpu.make_async_copy(trc.at[b], dst_c, wsem_c[b]).wait()


@jax.jit
def _rope_sc(grid, cos_h_all, sin_h_all, cos_w_all, sin_w_all):
    h_n, f = cos_h_all.shape
    w_n = cos_w_all.shape[0]
    ch = jnp.broadcast_to(cos_h_all[:, None, :], (h_n, w_n, f))
    cw = jnp.broadcast_to(cos_w_all[None, :, :], (h_n, w_n, f))
    sh = jnp.broadcast_to(sin_h_all[:, None, :], (h_n, w_n, f))
    sw = jnp.broadcast_to(sin_w_all[None, :, :], (h_n, w_n, f))
    tab = jnp.concatenate([sh, sw, sh, sw, ch, cw, ch, cw],
                          axis=-1).reshape(h_n * w_n, 2 * _D)
    h_flat = grid[..., 0].reshape(-1)
    w_flat = grid[..., 1].reshape(-1)

    mesh = plsc.VectorSubcoreMesh(core_axis_name="c", subcore_axis_name="s")
    ker = pl.kernel(
        _sc_body,
        out_type=[jax.ShapeDtypeStruct((_B, _D, _T), jnp.float32),
                  jax.ShapeDtypeStruct((_B, _D, _T), jnp.float32)],
        mesh=mesh,
        compiler_params=pltpu.CompilerParams(needs_layout_passes=False),
        scratch_types=[
            pltpu.VMEM((_CHUNK,), jnp.int32),
            pltpu.VMEM((_CHUNK,), jnp.int32),
            pltpu.VMEM((_NG, _GB), jnp.int32),
            pltpu.VMEM((_NB, _GB, 2 * _D), jnp.float32),
            pltpu.VMEM((_NB, _D, _GB), jnp.float32),
            pltpu.VMEM((_NB, _D, _GB), jnp.float32),
        ] + [pltpu.SemaphoreType.DMA] * (3 * _NB),
    )
    sin_t, cos_t = ker(h_flat, w_flat, tab)
    return jnp.swapaxes(sin_t, 1, 2), jnp.swapaxes(cos_t, 1, 2)


def kernel(grid, cos_h_all, sin_h_all, cos_w_all, sin_w_all):
    return _rope_sc(grid, cos_h_all, sin_h_all, cos_w_all, sin_w_all)


# speedup vs baseline: 1.9623x; 1.9623x over previous
import jax
import jax.numpy as jnp
from jax import lax
from jax.experimental import pallas as pl
from jax.experimental.pallas import tpu as pltpu
from jax.experimental.pallas import tpu_sc as plsc

_B = 64
_T = 1024
_N = _B * _T
_NC = 2
_NS = 16
_NW = _NC * _NS
_CHUNK = _N // _NW
_GB = 128
_NG = _CHUNK // _GB
_D = 128
_NB = 4
_NGRP = _NG // _NB


def _sc_body(h_hbm, w_hbm, tab_hbm, out_hbm, h_v, w_v, idx_v, rows, *sems):
    gsem = sems[:_NB]
    wsem = sems[_NB:]
    wid = lax.axis_index("s") * _NC + lax.axis_index("c")
    base = wid * _CHUNK
    pltpu.sync_copy(h_hbm.at[pl.ds(base, _CHUNK)], h_v)
    pltpu.sync_copy(w_hbm.at[pl.ds(base, _CHUNK)], w_v)

    def idx_body(j, carry):
        for k in range(_GB // 16):
            t0 = j * _GB + k * 16
            idx_v[j, pl.ds(k * 16, 16)] = h_v[pl.ds(t0, 16)] * 32 + w_v[pl.ds(t0, 16)]
        return carry

    lax.fori_loop(0, _NG, idx_body, 0)

    def fire_gather(j, b):
        pltpu.async_copy(tab_hbm.at[idx_v.at[j]], rows.at[b], gsem[b])

    for b in range(_NB):
        fire_gather(b, b)

    def group_body(g, carry):
        for b in range(_NB):
            j = g * _NB + b
            dst = out_hbm.at[pl.ds(base + j * _GB, _GB)]
            pltpu.make_async_copy(tab_hbm.at[idx_v.at[j]], rows.at[b],
                                  gsem[b]).wait()
            cw = pltpu.async_copy(rows.at[b], dst, wsem[b])

            @pl.when(g < _NGRP - 1)
            def _():
                cw.wait()
                fire_gather(j + _NB, b)

        return carry

    lax.fori_loop(0, _NGRP, group_body, 0)

    for b in range(_NB):
        j = (_NGRP - 1) * _NB + b
        pltpu.make_async_copy(rows.at[b],
                              out_hbm.at[pl.ds(base + j * _GB, _GB)],
                              wsem[b]).wait()


def _tc_body(x_ref, sin_ref, cos_ref):
    xt = jnp.swapaxes(x_ref[0], 0, 1)
    sin_ref[0] = xt[:64, :]
    cos_ref[0] = xt[64:, :]


@jax.jit
def _rope_sc(grid, cos_h_all, sin_h_all, cos_w_all, sin_w_all):
    h_n, f = cos_h_all.shape
    w_n = cos_w_all.shape[0]
    ch = jnp.broadcast_to(cos_h_all[:, None, :], (h_n, w_n, f))
    cw = jnp.broadcast_to(cos_w_all[None, :, :], (h_n, w_n, f))
    sh = jnp.broadcast_to(sin_h_all[:, None, :], (h_n, w_n, f))
    sw = jnp.broadcast_to(sin_w_all[None, :, :], (h_n, w_n, f))
    tab = jnp.concatenate([sh, sw, sh, sw, ch, cw, ch, cw],
                          axis=-1).reshape(h_n * w_n, _D)
    h_flat = grid[..., 0].reshape(-1)
    w_flat = grid[..., 1].reshape(-1)

    mesh = plsc.VectorSubcoreMesh(core_axis_name="c", subcore_axis_name="s")
    ker = pl.kernel(
        _sc_body,
        out_type=jax.ShapeDtypeStruct((_N, _D), jnp.float32),
        mesh=mesh,
        scratch_types=[
            pltpu.VMEM((_CHUNK,), jnp.int32),
            pltpu.VMEM((_CHUNK,), jnp.int32),
            pltpu.VMEM((_NG, _GB), jnp.int32),
            pltpu.VMEM((_NB, _GB, _D), jnp.float32),
        ] + [pltpu.SemaphoreType.DMA] * (2 * _NB),
    )
    comb = ker(h_flat, w_flat, tab)

    sin_t, cos_t = pl.pallas_call(
        _tc_body,
        grid=(_B,),
        in_specs=[pl.BlockSpec((1, _T, _D), lambda b: (b, 0, 0))],
        out_specs=[pl.BlockSpec((1, 64, _T), lambda b: (b, 0, 0)),
                   pl.BlockSpec((1, 64, _T), lambda b: (b, 0, 0))],
        out_shape=[jax.ShapeDtypeStruct((_B, 64, _T), jnp.float32),
                   jax.ShapeDtypeStruct((_B, 64, _T), jnp.float32)],
    )(comb.reshape(_B, _T, _D))

    return jnp.swapaxes(sin_t, 1, 2), jnp.swapaxes(cos_t, 1, 2)


def kernel(grid, cos_h_all, sin_h_all, cos_w_all, sin_w_all):
    return _rope_sc(grid, cos_h_all, sin_h_all, cos_w_all, sin_w_all)
